# baseline (speedup 1.0000x reference)
import functools

import jax
import jax.numpy as jnp
from jax import lax
from jax.experimental import pallas as pl
from jax.experimental.pallas import tpu as pltpu
from jax.experimental.pallas import tpu_sc as plsc

D = 64
NB = 16384
NF = 26
NC, NS = 2, 16
NW = NC * NS
SUB = 128
NJ = NB // SUB
NT = NF * NJ
T_PER_W = NT // NW
L = 16

_mesh = plsc.VectorSubcoreMesh(core_axis_name="c", subcore_axis_name="s")


@functools.partial(
    pl.kernel,
    mesh=_mesh,
    compiler_params=pltpu.CompilerParams(needs_layout_passes=False),
    out_type=jax.ShapeDtypeStruct((NF, D, NB), jnp.float32),
    scratch_types=[
        pltpu.VMEM((SUB,), jnp.int32),
        pltpu.VMEM((SUB,), jnp.int32),
        pltpu.VMEM((SUB,), jnp.int32),
        pltpu.VMEM((SUB,), jnp.int32),
        pltpu.VMEM((SUB, 2 * D), jnp.float32),
        pltpu.VMEM((SUB, 2 * D), jnp.float32),
        pltpu.VMEM((D, SUB), jnp.float32),
        pltpu.VMEM((D, SUB), jnp.float32),
        pltpu.SemaphoreType.DMA,
        pltpu.SemaphoreType.DMA,
        pltpu.SemaphoreType.DMA,
        pltpu.SemaphoreType.DMA,
    ],
)
def _emb_kernel(ctx_hbm, table_hbm, out_hbm, idx_a, idx_b, pidx_a, pidx_b,
                pbuf_a, pbuf_b, tbuf_a, tbuf_b, gsem_a, gsem_b, wsem_a, wsem_b):
    wid = lax.axis_index("s") * NC + lax.axis_index("c")
    t0 = wid * T_PER_W

    def stage(t, idx_v, pidx_v, pbuf, gsem):
        f = t >> 7
        j = t & (NJ - 1)
        pltpu.sync_copy(ctx_hbm.at[f, pl.ds(j * SUB, SUB)], idx_v)
        for m in range(SUB // L):
            pidx_v[pl.ds(m * L, L)] = lax.shift_right_logical(
                idx_v[pl.ds(m * L, L)], 1)
        pltpu.async_copy(table_hbm.at[pidx_v], pbuf, gsem)

    def extract(idx_v, pbuf, tbuf):
        for m in range(SUB // L):
            rows = jnp.arange(L, dtype=jnp.int32) + (m * L)
            h64 = lax.shift_left(
                lax.bitwise_and(idx_v[pl.ds(m * L, L)], 1), 6)
            for d in range(D):
                g = plsc.load_gather(pbuf, [rows, h64 + d])
                tbuf[d, pl.ds(m * L, L)] = g

    def write(t, tbuf, wsem):
        f = t >> 7
        j = t & (NJ - 1)
        return pltpu.async_copy(tbuf, out_hbm.at[f, :, pl.ds(j * SUB, SUB)],
                                wsem)

    def drain_g(pbuf, gsem):
        pltpu.make_async_copy(table_hbm.at[pl.ds(0, SUB)], pbuf, gsem).wait()

    def drain_w(tbuf, t, wsem):
        f = t >> 7
        j = t & (NJ - 1)
        pltpu.make_async_copy(tbuf, out_hbm.at[f, :, pl.ds(j * SUB, SUB)],
                              wsem).wait()

    stage(t0, idx_a, pidx_a, pbuf_a, gsem_a)
    stage(t0 + 1, idx_b, pidx_b, pbuf_b, gsem_b)

    def round_body(r, carry):
        ta = t0 + 2 * r
        tb = ta + 1
        drain_g(pbuf_a, gsem_a)

        @pl.when(r > 0)
        def _():
            drain_w(tbuf_a, ta - 2, wsem_a)
        extract(idx_a, pbuf_a, tbuf_a)

        @pl.when(2 * r + 2 < T_PER_W)
        def _():
            stage(ta + 2, idx_a, pidx_a, pbuf_a, gsem_a)
        write(ta, tbuf_a, wsem_a)
        drain_g(pbuf_b, gsem_b)

        @pl.when(r > 0)
        def _():
            drain_w(tbuf_b, tb - 2, wsem_b)
        extract(idx_b, pbuf_b, tbuf_b)

        @pl.when(2 * r + 3 < T_PER_W)
        def _():
            stage(tb + 2, idx_b, pidx_b, pbuf_b, gsem_b)
        write(tb, tbuf_b, wsem_b)
        return carry

    lax.fori_loop(0, T_PER_W // 2, round_body, 0)
    drain_w(tbuf_a, t0 + T_PER_W - 2, wsem_a)
    drain_w(tbuf_b, t0 + T_PER_W - 1, wsem_b)


def kernel(context, table):
    table2 = table.reshape(500000, 2 * D)
    ctx_t = context.T
    out = _emb_kernel(ctx_t, table2)
    return out.transpose(2, 0, 1)
